# baseline (speedup 1.0000x reference)
import functools

import jax
import jax.numpy as jnp
from jax import lax
from jax.experimental import pallas as pl
from jax.experimental.pallas import tpu as pltpu
from jax.experimental.pallas import tpu_sc as plsc

_NUM_CORES = 2
_NUM_SUBCORES = 16
_LANES = 16
_NW = _NUM_CORES * _NUM_SUBCORES

_N = 1000000
_D = 32
_LANE_T = 128
_SUB_T = 8
_NBANDS = _D // _SUB_T
_SH_TILES = (_N // _LANE_T) // _NW
_SH = _SH_TILES * _LANE_T
_BAND = _SH * _SUB_T
_TAIL0 = _SH * _NW
_TAILC = _N - _TAIL0
_MROWS = _NW * _NBANDS * _SUB_T
_ROWS = _MROWS + _NBANDS * _SUB_T
_TAILP = 640


def _build_detile():
    mesh = plsc.VectorSubcoreMesh(core_axis_name="c", subcore_axis_name="s")

    @functools.partial(
        pl.kernel,
        mesh=mesh,
        out_type=(jax.ShapeDtypeStruct((_ROWS, _SH), jnp.float32),
                  jax.ShapeDtypeStruct((_ROWS, _SH), jnp.float32)),
        compiler_params=pltpu.CompilerParams(needs_layout_passes=False),
        scratch_types=[
            pltpu.VMEM((_SUB_T, _TAILP), jnp.float32),
            pltpu.SemaphoreType.DMA,
        ],
    )
    def body(uembT, iembT, utail, itail, uflat, iflat, tailbuf, sem):
        wid = lax.axis_index("s") * _NUM_CORES + lax.axis_index("c")
        col0 = wid * _SH

        copies = []
        for src, dst in ((uembT, uflat), (iembT, iflat)):
            for k in range(_NBANDS):
                row0 = (wid * _NBANDS + k) * _SUB_T
                copies.append(pltpu.async_copy(
                    src.at[pl.ds(_SUB_T * k, _SUB_T), pl.ds(col0, _SH)],
                    dst.at[pl.ds(row0, _SUB_T), :], sem))
        for c in copies:
            c.wait()

        @pl.when(wid == 0)
        def _():
            for src, dst in ((utail, uflat), (itail, iflat)):
                for k in range(_NBANDS):
                    pltpu.async_copy(
                        src.at[pl.ds(_SUB_T * k, _SUB_T), :],
                        tailbuf, sem).wait()
                    pltpu.async_copy(
                        tailbuf,
                        dst.at[pl.ds(_MROWS + _SUB_T * k, _SUB_T),
                               pl.ds(0, _TAILP)], sem).wait()

    return body


def _build_lookup(B: int):
    bpw = B // _NW
    groups = bpw // _LANES
    mesh = plsc.VectorSubcoreMesh(core_axis_name="c", subcore_axis_name="s")

    @functools.partial(
        pl.kernel,
        mesh=mesh,
        out_type=jax.ShapeDtypeStruct((B,), jnp.float32),
        compiler_params=pltpu.CompilerParams(needs_layout_passes=False),
        scratch_types=[
            pltpu.VMEM((bpw,), jnp.int32),
            pltpu.VMEM((bpw,), jnp.int32),
            pltpu.VMEM((bpw * _D,), jnp.int32),
            pltpu.VMEM((bpw * _D,), jnp.int32),
            pltpu.VMEM((bpw * _D,), jnp.float32),
            pltpu.VMEM((bpw * _D,), jnp.float32),
            pltpu.VMEM((bpw,), jnp.float32),
            pltpu.VMEM((bpw,), jnp.float32),
            pltpu.VMEM((bpw,), jnp.float32),
            pltpu.SemaphoreType.DMA,
            pltpu.SemaphoreType.DMA,
            pltpu.SemaphoreType.DMA,
            pltpu.SemaphoreType.DMA,
        ],
    )
    def body(uid_hbm, iid_hbm, uflat_hbm, iflat_hbm, ubias_hbm, ibias_hbm,
             out_hbm, uid_v, iid_v, uix, iix, uval, ival, ub_v, ib_v,
             out_v, usem, isem, ubsem, ibsem):
        wid = lax.axis_index("s") * _NUM_CORES + lax.axis_index("c")
        base = wid * bpw

        pltpu.sync_copy(uid_hbm.at[pl.ds(base, bpw)], uid_v)
        pltpu.sync_copy(iid_hbm.at[pl.ds(base, bpw)], iid_v)

        for c in range(bpw // 128):
            cb = c * 128
            pltpu.async_copy(ubias_hbm.at[uid_v.at[pl.ds(cb, 128)]],
                             ub_v.at[pl.ds(cb, 128)], ubsem)
            pltpu.async_copy(ibias_hbm.at[iid_v.at[pl.ds(cb, 128)]],
                             ib_v.at[pl.ds(cb, 128)], ibsem)

        def gen(g, carry):
            gb = g * _LANES
            for ids, ixbuf in ((uid_v, uix), (iid_v, iix)):
                r = ids[pl.ds(gb, _LANES)]
                for d in range(_D):
                    ixbuf[pl.ds(d * bpw + gb, _LANES)] = r + d * _N
            return carry

        lax.fori_loop(0, groups, gen, 0)

        def fire(c, carry):
            cb = c * 128
            pltpu.async_copy(uflat_hbm.at[uix.at[pl.ds(cb, 128)]],
                             uval.at[pl.ds(cb, 128)], usem)
            pltpu.async_copy(iflat_hbm.at[iix.at[pl.ds(cb, 128)]],
                             ival.at[pl.ds(cb, 128)], isem)
            return carry

        lax.fori_loop(0, (bpw * _D) // 128, fire, 0)

        pltpu.make_async_copy(
            uflat_hbm.at[pl.ds(0, bpw * _D)], uval, usem).wait()
        pltpu.make_async_copy(
            iflat_hbm.at[pl.ds(0, bpw * _D)], ival, isem).wait()
        pltpu.make_async_copy(
            ubias_hbm.at[pl.ds(0, bpw)], ub_v, ubsem).wait()
        pltpu.make_async_copy(
            ibias_hbm.at[pl.ds(0, bpw)], ib_v, ibsem).wait()

        def group(g, carry):
            gb = g * _LANES
            acc = ub_v[pl.ds(gb, _LANES)] + ib_v[pl.ds(gb, _LANES)]
            for d in range(_D):
                acc = acc + (uval[pl.ds(d * bpw + gb, _LANES)]
                             * ival[pl.ds(d * bpw + gb, _LANES)])
            out_v[pl.ds(gb, _LANES)] = acc
            return carry

        lax.fori_loop(0, groups, group, 0)
        pltpu.sync_copy(out_v, out_hbm.at[pl.ds(base, bpw)])

    return body


@functools.lru_cache(maxsize=None)
def _build(B: int):
    return _build_lookup(B)


def kernel(user_id, item_id, user_emb, item_emb, user_bias, item_bias):
    B = user_id.shape[0]
    lookup = _build(B)
    return lookup(
        user_id.astype(jnp.int32),
        item_id.astype(jnp.int32),
        user_emb.T.reshape(-1),
        item_emb.T.reshape(-1),
        user_bias.reshape(-1),
        item_bias.reshape(-1),
    )


# speedup vs baseline: 5.7244x; 5.7244x over previous
import functools

import jax
import jax.numpy as jnp
from jax import lax
from jax.experimental import pallas as pl
from jax.experimental.pallas import tpu as pltpu
from jax.experimental.pallas import tpu_sc as plsc

_NUM_CORES = 2
_NUM_SUBCORES = 16
_LANES = 16
_NW = _NUM_CORES * _NUM_SUBCORES


@functools.lru_cache(maxsize=None)
def _build_sc_kernel(B: int, D: int):
    assert B % (_NW * _LANES) == 0
    bpw = B // _NW
    groups = bpw // _LANES

    mesh = plsc.VectorSubcoreMesh(core_axis_name="c", subcore_axis_name="s")

    @functools.partial(
        pl.kernel,
        mesh=mesh,
        out_type=jax.ShapeDtypeStruct((B,), jnp.float32),
        compiler_params=pltpu.CompilerParams(
            needs_layout_passes=False, use_tc_tiling_on_sc=False),
        scratch_types=[
            pltpu.VMEM((bpw,), jnp.int32),
            pltpu.VMEM((bpw,), jnp.int32),
            pltpu.VMEM((bpw, D), jnp.float32),
            pltpu.VMEM((bpw, D), jnp.float32),
            pltpu.VMEM((bpw,), jnp.float32),
            pltpu.VMEM((bpw,), jnp.float32),
            pltpu.VMEM((bpw,), jnp.float32),
            pltpu.SemaphoreType.DMA,
        ],
    )
    def body(uid_hbm, iid_hbm, uemb_hbm, iemb_hbm, ubias_hbm, ibias_hbm,
             out_hbm, uid_v, iid_v, urows, irows, ub_v, ib_v, out_v, sem):
        wid = lax.axis_index("s") * _NUM_CORES + lax.axis_index("c")
        base = wid * bpw

        pltpu.sync_copy(uid_hbm.at[pl.ds(base, bpw)], uid_v)
        pltpu.sync_copy(iid_hbm.at[pl.ds(base, bpw)], iid_v)

        c0 = pltpu.async_copy(uemb_hbm.at[uid_v], urows, sem)
        c1 = pltpu.async_copy(iemb_hbm.at[iid_v], irows, sem)
        c2 = pltpu.async_copy(ubias_hbm.at[uid_v], ub_v, sem)
        c3 = pltpu.async_copy(ibias_hbm.at[iid_v], ib_v, sem)
        c0.wait()
        c1.wait()
        c2.wait()
        c3.wait()

        def group(g, carry):
            gbase = g * _LANES
            rows = gbase + lax.iota(jnp.int32, _LANES)
            acc = ub_v[pl.ds(gbase, _LANES)] + ib_v[pl.ds(gbase, _LANES)]
            for d in range(D):
                col = jnp.full((_LANES,), d, jnp.int32)
                acc = acc + (plsc.load_gather(urows, [rows, col])
                             * plsc.load_gather(irows, [rows, col]))
            out_v[pl.ds(gbase, _LANES)] = acc
            return carry

        lax.fori_loop(0, groups, group, 0)
        pltpu.sync_copy(out_v, out_hbm.at[pl.ds(base, bpw)])

    return body


def kernel(user_id, item_id, user_emb, item_emb, user_bias, item_bias):
    B = user_id.shape[0]
    D = user_emb.shape[1]
    fn = _build_sc_kernel(B, D)
    return fn(
        user_id.astype(jnp.int32),
        item_id.astype(jnp.int32),
        user_emb,
        item_emb,
        user_bias.reshape(-1),
        item_bias.reshape(-1),
    )
